# Initial kernel scaffold; baseline (speedup 1.0000x reference)
import jax
import jax.numpy as jnp
from jax.experimental import pallas as pl


def kernel(src, dst, neg_dst, x, n_id, msg, t, edge_index, id_mapper, mem_table, last_update_table, W_msg, b_msg, W_self, W1, b1, W2, b2):
    raise NotImplementedError("write your pallas kernel here")


# speedup vs baseline: 2.9952x; 2.9952x over previous
import functools

import jax
import jax.numpy as jnp
from jax import lax
from jax.experimental import pallas as pl
from jax.experimental.pallas import tpu as pltpu
from jax.experimental.pallas import tpu_sc as plsc

NG = 100000
NSUB = 10000
NE = 160000
DM = 128
DF = 128
DMSG = 16
NBATCH = 2048
DZ = DM + DF
DH = 128

NC, NSC, LANES = 2, 16, 16
NW = NC * NSC
DHALF = DZ // NC

_f32 = jnp.float32
_i32 = jnp.int32


def _sc_mesh():
    return plsc.VectorSubcoreMesh(core_axis_name="c", subcore_axis_name="s",
                                  num_cores=NC, num_subcores=NSC)


_P0_K = 80
_P0_NBLK = NSUB // _P0_K


def _p0_body(nid_hbm, mem_hbm, x_hbm, m_hbm, xg_hbm, idx_v, mbuf, xbuf, sem):
    c = lax.axis_index("c")
    s = lax.axis_index("s")
    wid = s * NC + c
    nblk = jnp.where(wid < _P0_NBLK - (_P0_NBLK // NW) * NW,
                     _P0_NBLK // NW + 1, _P0_NBLK // NW)

    def blk(i, carry):
        r0 = (wid + NW * i) * _P0_K
        pltpu.sync_copy(nid_hbm.at[pl.ds(r0, _P0_K)], idx_v)
        pltpu.async_copy(mem_hbm.at[idx_v], mbuf, sem).wait()
        pltpu.sync_copy(mbuf, m_hbm.at[pl.ds(r0, _P0_K), :])
        pltpu.async_copy(x_hbm.at[idx_v], xbuf, sem).wait()
        pltpu.sync_copy(xbuf, xg_hbm.at[pl.ds(r0, _P0_K), :])
        return carry

    lax.fori_loop(0, nblk, blk, 0)


def _p0(n_id, mem_table, x):
    return pl.kernel(
        _p0_body,
        out_type=(jax.ShapeDtypeStruct((NSUB, DM), _f32),
                  jax.ShapeDtypeStruct((NSUB, DF), _f32)),
        mesh=_sc_mesh(),
        compiler_params=pltpu.CompilerParams(needs_layout_passes=False),
        scratch_types=[
            pltpu.VMEM((_P0_K,), _i32),
            pltpu.VMEM((_P0_K, DM), _f32),
            pltpu.VMEM((_P0_K, DF), _f32),
            pltpu.SemaphoreType.DMA,
        ],
    )(n_id, mem_table, x)


_P1S_K = 128
_P1S_NBLK = NE // _P1S_K


def _p1s_body(srce_hbm, t_hbm, nid_hbm, lut_hbm, relt_hbm,
              nid_tbl, lu_tbl, sidx, tbuf, rbuf):
    c = lax.axis_index("c")
    s = lax.axis_index("s")
    wid = s * NC + c
    pltpu.sync_copy(nid_hbm, nid_tbl)
    pltpu.sync_copy(lut_hbm, lu_tbl)
    per = _P1S_NBLK // NW
    nblk = jnp.where(wid < _P1S_NBLK - per * NW, per + 1, per)

    def blk(i, carry):
        e0 = (wid + NW * i) * _P1S_K
        pltpu.sync_copy(srce_hbm.at[pl.ds(e0, _P1S_K)], sidx)
        pltpu.sync_copy(t_hbm.at[pl.ds(e0, _P1S_K)], tbuf)
        for j in range(_P1S_K // LANES):
            sl = pl.ds(j * LANES, LANES)
            gi = plsc.load_gather(nid_tbl, [sidx[sl]])
            lu = plsc.load_gather(lu_tbl, [gi])
            rbuf[sl] = tbuf[sl] - lu
        pltpu.sync_copy(rbuf, relt_hbm.at[pl.ds(e0, _P1S_K)])
        return carry

    lax.fori_loop(0, nblk, blk, 0)


def _p1s(src_e, t, n_id, last_update_table):
    return pl.kernel(
        _p1s_body,
        out_type=jax.ShapeDtypeStruct((NE,), _f32),
        mesh=_sc_mesh(),
        compiler_params=pltpu.CompilerParams(needs_layout_passes=False),
        scratch_types=[
            pltpu.VMEM((NSUB,), _i32),
            pltpu.VMEM((NG,), _f32),
            pltpu.VMEM((_P1S_K,), _i32),
            pltpu.VMEM((_P1S_K,), _f32),
            pltpu.VMEM((_P1S_K,), _f32),
        ],
    )(src_e, t, n_id, last_update_table)


_P1T_BN = 2000


def _p1t_body(m_ref, xg_ref, a1, a2, s1, s2, zw_ref, zself_ref):
    mm = m_ref[...]
    xx = xg_ref[...]
    zw_ref[...] = mm @ a1[...] + xx @ a2[...]
    zself_ref[...] = mm @ s1[...] + xx @ s2[...]


def _p1t(m, xg, W_msg, W_self):
    nb = NSUB // _P1T_BN
    return pl.pallas_call(
        _p1t_body,
        grid=(NC, nb),
        in_specs=[
            pl.BlockSpec((_P1T_BN, DM), lambda c, i: (i, 0)),
            pl.BlockSpec((_P1T_BN, DF), lambda c, i: (i, 0)),
            pl.BlockSpec((DM, DHALF), lambda c, i: (0, c)),
            pl.BlockSpec((DF, DHALF), lambda c, i: (0, c)),
            pl.BlockSpec((DM, DHALF), lambda c, i: (0, c)),
            pl.BlockSpec((DF, DHALF), lambda c, i: (0, c)),
        ],
        out_specs=(
            pl.BlockSpec((_P1T_BN, DHALF), lambda c, i: (c * (NSUB // _P1T_BN) + i, 0)),
            pl.BlockSpec((_P1T_BN, DHALF), lambda c, i: (i, c)),
        ),
        out_shape=(jax.ShapeDtypeStruct((NC * NSUB, DHALF), _f32),
                   jax.ShapeDtypeStruct((NSUB, DZ), _f32)),
    )(m, xg, W_msg[:DM], W_msg[DM:DZ], W_self[:DM], W_self[DM:])


_P1E_BE = 2000


def _p1e_body(msg_ref, rt_ref, c_ref, wt_ref, bm_ref, eb_ref):
    eb_ref[...] = (msg_ref[...] @ c_ref[...]
                   + rt_ref[...] * wt_ref[...] + bm_ref[...])


def _p1e(msg, rel_t, W_msg, b_msg):
    ne = NE // _P1E_BE
    return pl.pallas_call(
        _p1e_body,
        grid=(NC, ne),
        in_specs=[
            pl.BlockSpec((_P1E_BE, DMSG), lambda c, i: (i, 0)),
            pl.BlockSpec((_P1E_BE, 1), lambda c, i: (i, 0)),
            pl.BlockSpec((DMSG, DHALF), lambda c, i: (0, c)),
            pl.BlockSpec((1, DHALF), lambda c, i: (0, c)),
            pl.BlockSpec((1, DHALF), lambda c, i: (0, c)),
        ],
        out_specs=pl.BlockSpec((_P1E_BE, DHALF),
                               lambda c, i: (c * (NE // _P1E_BE) + i, 0)),
        out_shape=jax.ShapeDtypeStruct((NC * NE, DHALF), _f32),
    )(msg, rel_t.reshape(NE, 1), W_msg[DZ:DZ + DMSG],
      W_msg[DZ + DMSG].reshape(1, DZ), b_msg.reshape(1, DZ))


_P2_K = 128
_P2_NBLK = NE // _P2_K
_P2_PER = _P2_NBLK // NSC
_P2_R0 = 624
_P2_CHUNKS = 5


def _p2_body(zw_hbm, eb_hbm, srce_hbm, dste_hbm, agg_hbm,
             sidx, didx, buf, aggsh, sem):
    c = lax.axis_index("c")
    s = lax.axis_index("s")

    def zrow(r, carry):
        for j in range(DHALF // LANES):
            buf[r, pl.ds(j * LANES, LANES)] = jnp.zeros((LANES,), _f32)
        return carry

    lax.fori_loop(0, _P2_K, zrow, 0)
    r0 = s * _P2_R0
    for k in range(_P2_CHUNKS):
        pltpu.sync_copy(buf, aggsh.at[pl.ds(r0 + k * _P2_K, _P2_K), :])
    plsc.subcore_barrier()

    nblk = jnp.where(s < _P2_NBLK - _P2_PER * NSC, _P2_PER + 1, _P2_PER)
    zoff = c * NSUB

    def blk(i, carry):
        e0 = (s + NSC * i) * _P2_K
        pltpu.sync_copy(srce_hbm.at[pl.ds(e0, _P2_K)], sidx)
        for j in range(_P2_K // LANES):
            sl = pl.ds(j * LANES, LANES)
            sidx[sl] = sidx[sl] + zoff
        pltpu.sync_copy(dste_hbm.at[pl.ds(e0, _P2_K)], didx)
        pltpu.sync_copy(eb_hbm.at[pl.ds(c * NE + e0, _P2_K), :], buf)
        pltpu.async_copy(zw_hbm.at[sidx], buf, sem, add=True).wait()

        def rrow(r, carry2):
            for j in range(DHALF // LANES):
                sl2 = pl.ds(j * LANES, LANES)
                buf[r, sl2] = jnp.maximum(buf[r, sl2], 0.0)
            return carry2

        lax.fori_loop(0, _P2_K, rrow, 0)
        pltpu.sync_copy(buf, aggsh.at[didx], add=True)
        return carry

    lax.fori_loop(0, nblk, blk, 0)
    plsc.subcore_barrier()

    for k in range(_P2_CHUNKS):
        pltpu.sync_copy(aggsh.at[pl.ds(r0 + k * _P2_K, _P2_K), :], buf)
        pltpu.sync_copy(buf, agg_hbm.at[pl.ds(c * NSUB + r0 + k * _P2_K, _P2_K), :])


def _p2(zw_st, eb_st, src_e, dst_e):
    return pl.kernel(
        _p2_body,
        out_type=jax.ShapeDtypeStruct((NC * NSUB, DHALF), _f32),
        mesh=_sc_mesh(),
        compiler_params=pltpu.CompilerParams(needs_layout_passes=False),
        scratch_types=[
            pltpu.VMEM((_P2_K,), _i32),
            pltpu.VMEM((_P2_K,), _i32),
            pltpu.VMEM((_P2_K, DHALF), _f32),
            pltpu.VMEM_SHARED((NSUB, DHALF), _f32),
            pltpu.SemaphoreType.DMA,
        ],
    )(zw_st, eb_st, src_e, dst_e)


_P3_BN = 2000


def _p3_body(zself_ref, agg_ref, znew_ref):
    znew_ref[...] = jnp.maximum(zself_ref[...] + agg_ref[...], 0.0)


def _p3(zself, agg_st):
    nb = NSUB // _P3_BN
    return pl.pallas_call(
        _p3_body,
        grid=(NC, nb),
        in_specs=[
            pl.BlockSpec((_P3_BN, DHALF), lambda c, i: (i, c)),
            pl.BlockSpec((_P3_BN, DHALF), lambda c, i: (c * (NSUB // _P3_BN) + i, 0)),
        ],
        out_specs=pl.BlockSpec((_P3_BN, DHALF), lambda c, i: (i, c)),
        out_shape=jax.ShapeDtypeStruct((NSUB, DZ), _f32),
    )(zself, agg_st)


_P4_PB = NBATCH // NW
_P4_H = 32


def _p4_body(src_hbm, dst_hbm, nd_hbm, idmap_hbm, znew_hbm, m_hbm,
             zs_hbm, zd_hbm, znd_hbm, ms_hbm, md_hbm,
             idmap_tbl, iidx, gbuf, zrows, mrows, sem):
    c = lax.axis_index("c")
    s = lax.axis_index("s")
    wid = s * NC + c
    pltpu.sync_copy(idmap_hbm, idmap_tbl)

    def do(idx_hbm, zout, mout):
        for h in range(_P4_PB // _P4_H):
            b0 = wid * _P4_PB + h * _P4_H
            pltpu.sync_copy(idx_hbm.at[pl.ds(b0, _P4_H)], iidx)
            for j in range(_P4_H // LANES):
                sl = pl.ds(j * LANES, LANES)
                gbuf[sl] = plsc.load_gather(idmap_tbl, [iidx[sl]])
            pltpu.async_copy(znew_hbm.at[gbuf], zrows, sem).wait()
            pltpu.sync_copy(zrows, zout.at[pl.ds(b0, _P4_H), :])
            if mout is not None:
                pltpu.async_copy(m_hbm.at[gbuf], mrows, sem).wait()
                pltpu.sync_copy(mrows, mout.at[pl.ds(b0, _P4_H), :])

    do(src_hbm, zs_hbm, ms_hbm)
    do(dst_hbm, zd_hbm, md_hbm)
    do(nd_hbm, znd_hbm, None)


def _p4(src, dst, neg_dst, id_mapper, znew, m):
    return pl.kernel(
        _p4_body,
        out_type=(jax.ShapeDtypeStruct((NBATCH, DZ), _f32),
                  jax.ShapeDtypeStruct((NBATCH, DZ), _f32),
                  jax.ShapeDtypeStruct((NBATCH, DZ), _f32),
                  jax.ShapeDtypeStruct((NBATCH, DM), _f32),
                  jax.ShapeDtypeStruct((NBATCH, DM), _f32)),
        mesh=_sc_mesh(),
        compiler_params=pltpu.CompilerParams(needs_layout_passes=False),
        scratch_types=[
            pltpu.VMEM((NG,), _i32),
            pltpu.VMEM((_P4_H,), _i32),
            pltpu.VMEM((_P4_H,), _i32),
            pltpu.VMEM((_P4_H, DZ), _f32),
            pltpu.VMEM((_P4_H, DM), _f32),
            pltpu.SemaphoreType.DMA,
        ],
    )(src, dst, neg_dst, id_mapper, znew, m)


def _p5_body(zs_ref, zd_ref, znd_ref, w1t, w1b, b1_ref, w2_ref, b2_ref,
             pos_ref, neg_ref):
    zs = zs_ref[...]
    b2 = b2_ref[0, 0]
    hp = jnp.maximum(zs @ w1t[...] + zd_ref[...] @ w1b[...] + b1_ref[...], 0.0)
    pos_ref[...] = hp @ w2_ref[...] + b2
    hn = jnp.maximum(zs @ w1t[...] + znd_ref[...] @ w1b[...] + b1_ref[...], 0.0)
    neg_ref[...] = hn @ w2_ref[...] + b2


def _p5(zs, zd, znd, W1, b1, W2, b2):
    return pl.pallas_call(
        _p5_body,
        out_shape=(jax.ShapeDtypeStruct((NBATCH, 1), _f32),
                   jax.ShapeDtypeStruct((NBATCH, 1), _f32)),
    )(zs, zd, znd, W1[:DZ], W1[DZ:], b1.reshape(1, DH), W2,
      b2.reshape(1, 1))


def kernel(src, dst, neg_dst, x, n_id, msg, t, edge_index, id_mapper,
           mem_table, last_update_table, W_msg, b_msg, W_self, W1, b1, W2, b2):
    src_e = edge_index[0]
    dst_e = edge_index[1]
    m, xg = _p0(n_id, mem_table, x)
    rel_t = _p1s(src_e, t, n_id, last_update_table)
    zw_st, zself = _p1t(m, xg, W_msg, W_self)
    eb_st = _p1e(msg, rel_t, W_msg, b_msg)
    agg_st = _p2(zw_st, eb_st, src_e, dst_e)
    znew = _p3(zself, agg_st)
    zs, zd, znd, ms, md = _p4(src, dst, neg_dst, id_mapper, znew, m)
    pos, neg = _p5(zs, zd, znd, W1, b1, W2, b2)
    return pos, neg, ms, md


# speedup vs baseline: 3.0045x; 1.0031x over previous
import jax
import jax.numpy as jnp
from jax import lax
from jax.experimental import pallas as pl
from jax.experimental.pallas import tpu as pltpu
from jax.experimental.pallas import tpu_sc as plsc

NG = 100000
NSUB = 10000
NE = 160000
DM = 128
DF = 128
DMSG = 16
NBATCH = 2048
DZ = DM + DF
DH = 128

NC, NSC, LANES = 2, 16, 16
NW = NC * NSC
DHALF = DZ // NC

_f32 = jnp.float32
_i32 = jnp.int32


def _sc_mesh():
    return plsc.VectorSubcoreMesh(core_axis_name="c", subcore_axis_name="s",
                                  num_cores=NC, num_subcores=NSC)


_P0_K = 80
_P0_NBLK = NSUB // _P0_K


def _p0_body(nid_hbm, mem_hbm, x_hbm, lut_hbm, m_hbm, xg_hbm, lu_hbm,
             idx_v, mbuf, xbuf, lubuf, lut_tbl, sem):
    c = lax.axis_index("c")
    s = lax.axis_index("s")
    wid = s * NC + c
    pltpu.sync_copy(lut_hbm, lut_tbl)
    nblk = jnp.where(wid < _P0_NBLK - (_P0_NBLK // NW) * NW,
                     _P0_NBLK // NW + 1, _P0_NBLK // NW)

    def blk(i, carry):
        r0 = (wid + NW * i) * _P0_K
        pltpu.sync_copy(nid_hbm.at[pl.ds(r0, _P0_K)], idx_v)
        pltpu.async_copy(mem_hbm.at[idx_v], mbuf, sem).wait()
        pltpu.sync_copy(mbuf, m_hbm.at[pl.ds(r0, _P0_K), :])
        pltpu.async_copy(x_hbm.at[idx_v], xbuf, sem).wait()
        pltpu.sync_copy(xbuf, xg_hbm.at[pl.ds(r0, _P0_K), :])
        for j in range(_P0_K // LANES):
            sl = pl.ds(j * LANES, LANES)
            lubuf[sl] = plsc.load_gather(lut_tbl, [idx_v[sl]])
        pltpu.sync_copy(lubuf, lu_hbm.at[pl.ds(r0, _P0_K)])
        return carry

    lax.fori_loop(0, nblk, blk, 0)


def _p0(n_id, mem_table, x, last_update_table):
    return pl.kernel(
        _p0_body,
        out_type=(jax.ShapeDtypeStruct((NSUB, DM), _f32),
                  jax.ShapeDtypeStruct((NSUB, DF), _f32),
                  jax.ShapeDtypeStruct((NSUB,), _f32)),
        mesh=_sc_mesh(),
        compiler_params=pltpu.CompilerParams(needs_layout_passes=False),
        scratch_types=[
            pltpu.VMEM((_P0_K,), _i32),
            pltpu.VMEM((_P0_K, DM), _f32),
            pltpu.VMEM((_P0_K, DF), _f32),
            pltpu.VMEM((_P0_K,), _f32),
            pltpu.VMEM((NG,), _f32),
            pltpu.SemaphoreType.DMA,
        ],
    )(n_id, mem_table, x, last_update_table)


_P1T_BN = 2000


def _p1t_body(m_ref, xg_ref, lu_ref, a1, a2, wt_ref, s1, s2, zw_ref, zself_ref):
    mm = m_ref[...]
    xx = xg_ref[...]
    zw_ref[...] = mm @ a1[...] + xx @ a2[...] - lu_ref[...] * wt_ref[...]
    zself_ref[...] = mm @ s1[...] + xx @ s2[...]


def _p1t(m, xg, lu2, W_msg, W_self):
    nb = NSUB // _P1T_BN
    return pl.pallas_call(
        _p1t_body,
        grid=(NC, nb),
        in_specs=[
            pl.BlockSpec((_P1T_BN, DM), lambda c, i: (i, 0)),
            pl.BlockSpec((_P1T_BN, DF), lambda c, i: (i, 0)),
            pl.BlockSpec((_P1T_BN, 1), lambda c, i: (i, 0)),
            pl.BlockSpec((DM, DHALF), lambda c, i: (0, c)),
            pl.BlockSpec((DF, DHALF), lambda c, i: (0, c)),
            pl.BlockSpec((1, DHALF), lambda c, i: (0, c)),
            pl.BlockSpec((DM, DHALF), lambda c, i: (0, c)),
            pl.BlockSpec((DF, DHALF), lambda c, i: (0, c)),
        ],
        out_specs=(
            pl.BlockSpec((_P1T_BN, DHALF), lambda c, i: (c * (NSUB // _P1T_BN) + i, 0)),
            pl.BlockSpec((_P1T_BN, DHALF), lambda c, i: (i, c)),
        ),
        out_shape=(jax.ShapeDtypeStruct((NC * NSUB, DHALF), _f32),
                   jax.ShapeDtypeStruct((NSUB, DZ), _f32)),
    )(m, xg, lu2, W_msg[:DM], W_msg[DM:DZ], W_msg[DZ + DMSG].reshape(1, DZ),
      W_self[:DM], W_self[DM:])


_P1E_BE = 2000


def _p1e_body(msg_ref, rt_ref, c_ref, wt_ref, bm_ref, eb_ref):
    eb_ref[...] = (msg_ref[...] @ c_ref[...]
                   + rt_ref[...] * wt_ref[...] + bm_ref[...])


def _p1e(msg, t, W_msg, b_msg):
    ne = NE // _P1E_BE
    return pl.pallas_call(
        _p1e_body,
        grid=(NC, ne),
        in_specs=[
            pl.BlockSpec((_P1E_BE, DMSG), lambda c, i: (i, 0)),
            pl.BlockSpec((_P1E_BE, 1), lambda c, i: (i, 0)),
            pl.BlockSpec((DMSG, DHALF), lambda c, i: (0, c)),
            pl.BlockSpec((1, DHALF), lambda c, i: (0, c)),
            pl.BlockSpec((1, DHALF), lambda c, i: (0, c)),
        ],
        out_specs=pl.BlockSpec((_P1E_BE, DHALF),
                               lambda c, i: (c * (NE // _P1E_BE) + i, 0)),
        out_shape=jax.ShapeDtypeStruct((NC * NE, DHALF), _f32),
    )(msg, t.reshape(NE, 1), W_msg[DZ:DZ + DMSG],
      W_msg[DZ + DMSG].reshape(1, DZ), b_msg.reshape(1, DZ))


_P2_K = 128
_P2_NBLK = NE // _P2_K
_P2_PER = _P2_NBLK // NSC
_P2_R0 = 624
_P2_CHUNKS = 5


def _p2_body(zw_hbm, eb_hbm, srce_hbm, dste_hbm, agg_hbm,
             sidx, didx, buf, aggsh, sem):
    c = lax.axis_index("c")
    s = lax.axis_index("s")

    def zrow(r, carry):
        for j in range(DHALF // LANES):
            buf[r, pl.ds(j * LANES, LANES)] = jnp.zeros((LANES,), _f32)
        return carry

    lax.fori_loop(0, _P2_K, zrow, 0)
    r0 = s * _P2_R0
    for k in range(_P2_CHUNKS):
        pltpu.sync_copy(buf, aggsh.at[pl.ds(r0 + k * _P2_K, _P2_K), :])
    plsc.subcore_barrier()

    nblk = jnp.where(s < _P2_NBLK - _P2_PER * NSC, _P2_PER + 1, _P2_PER)
    zoff = c * NSUB

    def blk(i, carry):
        e0 = (s + NSC * i) * _P2_K
        pltpu.sync_copy(srce_hbm.at[pl.ds(e0, _P2_K)], sidx)
        for j in range(_P2_K // LANES):
            sl = pl.ds(j * LANES, LANES)
            sidx[sl] = sidx[sl] + zoff
        pltpu.sync_copy(dste_hbm.at[pl.ds(e0, _P2_K)], didx)
        pltpu.sync_copy(eb_hbm.at[pl.ds(c * NE + e0, _P2_K), :], buf)
        pltpu.async_copy(zw_hbm.at[sidx], buf, sem, add=True).wait()

        def rrow(r, carry2):
            for j in range(DHALF // LANES):
                sl2 = pl.ds(j * LANES, LANES)
                buf[r, sl2] = jnp.maximum(buf[r, sl2], 0.0)
            return carry2

        lax.fori_loop(0, _P2_K, rrow, 0)
        pltpu.sync_copy(buf, aggsh.at[didx], add=True)
        return carry

    lax.fori_loop(0, nblk, blk, 0)
    plsc.subcore_barrier()

    for k in range(_P2_CHUNKS):
        pltpu.sync_copy(aggsh.at[pl.ds(r0 + k * _P2_K, _P2_K), :], buf)
        pltpu.sync_copy(buf, agg_hbm.at[pl.ds(c * NSUB + r0 + k * _P2_K, _P2_K), :])


def _p2(zw_st, eb_st, src_e, dst_e):
    return pl.kernel(
        _p2_body,
        out_type=jax.ShapeDtypeStruct((NC * NSUB, DHALF), _f32),
        mesh=_sc_mesh(),
        compiler_params=pltpu.CompilerParams(needs_layout_passes=False),
        scratch_types=[
            pltpu.VMEM((_P2_K,), _i32),
            pltpu.VMEM((_P2_K,), _i32),
            pltpu.VMEM((_P2_K, DHALF), _f32),
            pltpu.VMEM_SHARED((NSUB, DHALF), _f32),
            pltpu.SemaphoreType.DMA,
        ],
    )(zw_st, eb_st, src_e, dst_e)


_P3_BN = 2000


def _p3_body(zself_ref, agg_ref, znew_ref):
    znew_ref[...] = jnp.maximum(zself_ref[...] + agg_ref[...], 0.0)


def _p3(zself, agg_st):
    nb = NSUB // _P3_BN
    return pl.pallas_call(
        _p3_body,
        grid=(NC, nb),
        in_specs=[
            pl.BlockSpec((_P3_BN, DHALF), lambda c, i: (i, c)),
            pl.BlockSpec((_P3_BN, DHALF), lambda c, i: (c * (NSUB // _P3_BN) + i, 0)),
        ],
        out_specs=pl.BlockSpec((_P3_BN, DHALF), lambda c, i: (i, c)),
        out_shape=jax.ShapeDtypeStruct((NSUB, DZ), _f32),
    )(zself, agg_st)


_P4_PB = NBATCH // NW
_P4_H = 32


def _p4_body(src_hbm, dst_hbm, nd_hbm, idmap_hbm, znew_hbm, m_hbm,
             zs_hbm, zd_hbm, znd_hbm, ms_hbm, md_hbm,
             idmap_tbl, iidx, gbuf, zrows, mrows, sem):
    c = lax.axis_index("c")
    s = lax.axis_index("s")
    wid = s * NC + c
    pltpu.sync_copy(idmap_hbm, idmap_tbl)

    def do(idx_hbm, zout, mout):
        for h in range(_P4_PB // _P4_H):
            b0 = wid * _P4_PB + h * _P4_H
            pltpu.sync_copy(idx_hbm.at[pl.ds(b0, _P4_H)], iidx)
            for j in range(_P4_H // LANES):
                sl = pl.ds(j * LANES, LANES)
                gbuf[sl] = plsc.load_gather(idmap_tbl, [iidx[sl]])
            pltpu.async_copy(znew_hbm.at[gbuf], zrows, sem).wait()
            pltpu.sync_copy(zrows, zout.at[pl.ds(b0, _P4_H), :])
            if mout is not None:
                pltpu.async_copy(m_hbm.at[gbuf], mrows, sem).wait()
                pltpu.sync_copy(mrows, mout.at[pl.ds(b0, _P4_H), :])

    do(src_hbm, zs_hbm, ms_hbm)
    do(dst_hbm, zd_hbm, md_hbm)
    do(nd_hbm, znd_hbm, None)


def _p4(src, dst, neg_dst, id_mapper, znew, m):
    return pl.kernel(
        _p4_body,
        out_type=(jax.ShapeDtypeStruct((NBATCH, DZ), _f32),
                  jax.ShapeDtypeStruct((NBATCH, DZ), _f32),
                  jax.ShapeDtypeStruct((NBATCH, DZ), _f32),
                  jax.ShapeDtypeStruct((NBATCH, DM), _f32),
                  jax.ShapeDtypeStruct((NBATCH, DM), _f32)),
        mesh=_sc_mesh(),
        compiler_params=pltpu.CompilerParams(needs_layout_passes=False),
        scratch_types=[
            pltpu.VMEM((NG,), _i32),
            pltpu.VMEM((_P4_H,), _i32),
            pltpu.VMEM((_P4_H,), _i32),
            pltpu.VMEM((_P4_H, DZ), _f32),
            pltpu.VMEM((_P4_H, DM), _f32),
            pltpu.SemaphoreType.DMA,
        ],
    )(src, dst, neg_dst, id_mapper, znew, m)


def _p5_body(zs_ref, zd_ref, znd_ref, w1t, w1b, b1_ref, w2_ref, b2_ref,
             pos_ref, neg_ref):
    zs = zs_ref[...]
    b2 = b2_ref[0, 0]
    hp = jnp.maximum(zs @ w1t[...] + zd_ref[...] @ w1b[...] + b1_ref[...], 0.0)
    pos_ref[...] = hp @ w2_ref[...] + b2
    hn = jnp.maximum(zs @ w1t[...] + znd_ref[...] @ w1b[...] + b1_ref[...], 0.0)
    neg_ref[...] = hn @ w2_ref[...] + b2


def _p5(zs, zd, znd, W1, b1, W2, b2):
    return pl.pallas_call(
        _p5_body,
        out_shape=(jax.ShapeDtypeStruct((NBATCH, 1), _f32),
                   jax.ShapeDtypeStruct((NBATCH, 1), _f32)),
    )(zs, zd, znd, W1[:DZ], W1[DZ:], b1.reshape(1, DH), W2,
      b2.reshape(1, 1))


def kernel(src, dst, neg_dst, x, n_id, msg, t, edge_index, id_mapper,
           mem_table, last_update_table, W_msg, b_msg, W_self, W1, b1, W2, b2):
    src_e = edge_index[0]
    dst_e = edge_index[1]
    m, xg, lu = _p0(n_id, mem_table, x, last_update_table)
    zw_st, zself = _p1t(m, xg, lu.reshape(NSUB, 1), W_msg, W_self)
    eb_st = _p1e(msg, t, W_msg, b_msg)
    agg_st = _p2(zw_st, eb_st, src_e, dst_e)
    znew = _p3(zself, agg_st)
    zs, zd, znd, ms, md = _p4(src, dst, neg_dst, id_mapper, znew, m)
    pos, neg = _p5(zs, zd, znd, W1, b1, W2, b2)
    return pos, neg, ms, md
